# baseline (speedup 1.0000x reference)
import functools

import jax
import jax.numpy as jnp
from jax import lax
from jax.experimental import pallas as pl
from jax.experimental.pallas import tpu as pltpu
from jax.experimental.pallas import tpu_sc as plsc

_KC = 1280
_TAIL = 256
_LANES = 16


def _mv_body(x_hbm, xt_hbm, w_hbm, wt_hbm, b_hbm, o_hbm,
             xb0, xb1, tb, wb0, wb1, wtb, acc, res, bv,
             sem0, sem1, tsem, bsem, *, rows, nkc, nc):
    wid = lax.axis_index("s") * nc + lax.axis_index("c")
    base = wid * rows
    xbufs = (xb0, xb1)
    wbufs = (wb0, wb1)
    sems = (sem0, sem1)

    def x_copy(k, slot):
        return pltpu.make_async_copy(
            x_hbm.at[pl.ds(base, rows), pl.ds(k * _KC, _KC)],
            xbufs[slot],
            sems[slot],
        )

    def w_copy(k, slot):
        return pltpu.make_async_copy(
            w_hbm.at[pl.ds(k * _KC, _KC)],
            wbufs[slot],
            sems[slot],
        )

    xt_copy = pltpu.make_async_copy(
        xt_hbm.at[pl.ds(base, rows), pl.ds(0, _TAIL)], tb, tsem)
    wt_copy = pltpu.make_async_copy(wt_hbm, wtb, tsem)
    b_copy = pltpu.make_async_copy(b_hbm, bv, bsem)

    b_copy.start()
    xt_copy.start()
    wt_copy.start()
    for b in range(2):
        x_copy(b, b).start()
        w_copy(b, b).start()

    def zero_acc(r, _):
        acc[r] = jnp.zeros((_LANES,), jnp.float32)
        return 0

    lax.fori_loop(0, rows, zero_acc, 0)

    def accum_chunk(xb, wb, ncols):
        for cb in range(ncols // (_LANES * _LANES)):
            ws = [wb[pl.ds((cb * _LANES + c) * _LANES, _LANES)]
                  for c in range(_LANES)]

            def row_body(r, _):
                a = acc[r]
                for c in range(_LANES):
                    a += xb[r, pl.ds((cb * _LANES + c) * _LANES, _LANES)] * ws[c]
                acc[r] = a
                return 0

            lax.fori_loop(0, rows, row_body, 0)

    def outer(g, _):
        for b in range(2):
            k = 2 * g + b
            x_copy(k, b).wait()
            w_copy(k, b).wait()
            accum_chunk(xbufs[b], wbufs[b], _KC)

            nxt = k + 2

            @pl.when(nxt < nkc)
            def _():
                x_copy(nxt, b).start()
                w_copy(nxt, b).start()
        return 0

    lax.fori_loop(0, nkc // 2, outer, 0)

    xt_copy.wait()
    wt_copy.wait()
    accum_chunk(tb, wtb, _TAIL)

    b_copy.wait()
    lanes = lax.iota(jnp.int32, _LANES)

    def shuffle(v, idx):
        return lax.gather(
            v, idx[:, None],
            lax.GatherDimensionNumbers(
                offset_dims=(), collapsed_slice_dims=(0,),
                start_index_map=(0,)),
            slice_sizes=(1,),
            mode=lax.GatherScatterMode.PROMISE_IN_BOUNDS)

    def lane_sum(v):
        for s in (8, 4, 2, 1):
            v = v + shuffle(v, jnp.bitwise_xor(lanes, s))
        return v

    bias_full = shuffle(bv[...], jnp.bitwise_xor(lanes, lanes))
    for g in range(rows // _LANES):
        resv = jnp.zeros((_LANES,), jnp.float32)
        for j in range(_LANES):
            resv = jnp.where(lanes == j, lane_sum(acc[g * _LANES + j]), resv)
        res[pl.ds(g * _LANES, _LANES)] = resv + bias_full
    pltpu.sync_copy(res, o_hbm.at[pl.ds(base, rows)])


@jax.jit
def kernel(x, weight, bias):
    batch, vocab = x.shape
    info = plsc.get_sparse_core_info()
    nc, ns = info.num_cores, info.num_subcores
    nw = nc * ns
    rows = batch // nw
    nkc = vocab // _KC
    head = nkc * _KC
    tail = vocab - head

    wflat = weight.reshape(-1)
    xt = jnp.pad(x[:, head:], ((0, 0), (0, _TAIL - tail)))
    wt = jnp.pad(wflat[head:], (0, _TAIL - tail))
    bpad = jnp.pad(bias.reshape(-1), (0, _LANES - 1))

    mesh = plsc.VectorSubcoreMesh(core_axis_name="c", subcore_axis_name="s")
    body = functools.partial(_mv_body, rows=rows, nkc=nkc, nc=nc)
    fn = pl.kernel(
        body,
        mesh=mesh,
        out_type=jax.ShapeDtypeStruct((batch,), jnp.float32),
        scratch_types=[
            pltpu.VMEM((rows, _KC), jnp.float32),
            pltpu.VMEM((rows, _KC), jnp.float32),
            pltpu.VMEM((rows, _TAIL), jnp.float32),
            pltpu.VMEM((_KC,), jnp.float32),
            pltpu.VMEM((_KC,), jnp.float32),
            pltpu.VMEM((_TAIL,), jnp.float32),
            pltpu.VMEM((rows, _LANES), jnp.float32),
            pltpu.VMEM((rows,), jnp.float32),
            pltpu.VMEM((_LANES,), jnp.float32),
            pltpu.SemaphoreType.DMA,
            pltpu.SemaphoreType.DMA,
            pltpu.SemaphoreType.DMA,
            pltpu.SemaphoreType.DMA,
        ],
    )
    out = fn(x, xt, wflat[:head], wt, bpad)
    return out.reshape(batch, 1)


# speedup vs baseline: 1.0720x; 1.0720x over previous
import functools

import jax
import jax.numpy as jnp
from jax import lax
from jax.experimental import pallas as pl
from jax.experimental.pallas import tpu as pltpu
from jax.experimental.pallas import tpu_sc as plsc

_KC = 1280
_TAIL = 256
_LANES = 16


def _mv_body(x_hbm, xt_hbm, w_hbm, wt_hbm, b_hbm, o_hbm,
             xb0, xb1, tb, wb0, wb1, wtb, acc, res, bv,
             sem0, sem1, tsem, bsem, *, rows, nkc, nc):
    wid = lax.axis_index("s") * nc + lax.axis_index("c")
    base = wid * rows
    xbufs = (xb0, xb1)
    wbufs = (wb0, wb1)
    sems = (sem0, sem1)

    def x_copy(k, slot):
        return pltpu.make_async_copy(
            x_hbm.at[pl.ds(base, rows), pl.ds(k * _KC, _KC)],
            xbufs[slot],
            sems[slot],
        )

    def w_copy(k, slot):
        return pltpu.make_async_copy(
            w_hbm.at[pl.ds(k * _KC, _KC)],
            wbufs[slot],
            sems[slot],
        )

    xt_copy = pltpu.make_async_copy(
        xt_hbm.at[pl.ds(base, rows), pl.ds(0, _TAIL)], tb, tsem)
    wt_copy = pltpu.make_async_copy(wt_hbm, wtb, tsem)
    b_copy = pltpu.make_async_copy(b_hbm, bv, bsem)

    b_copy.start()
    xt_copy.start()
    wt_copy.start()
    for b in range(2):
        x_copy(b, b).start()
        w_copy(b, b).start()

    def zero_acc(r, _):
        acc[r] = jnp.zeros((_LANES,), jnp.float32)
        return 0

    lax.fori_loop(0, rows, zero_acc, 0)

    def accum_chunk(xb, wb, ncols):
        def row_body(r, _):
            a = acc[r]
            for c in range(ncols // _LANES):
                a += (xb[r, pl.ds(c * _LANES, _LANES)]
                      * wb[pl.ds(c * _LANES, _LANES)])
            acc[r] = a
            return 0

        lax.fori_loop(0, rows, row_body, 0)

    def outer(g, _):
        for b in range(2):
            k = 2 * g + b
            x_copy(k, b).wait()
            w_copy(k, b).wait()
            accum_chunk(xbufs[b], wbufs[b], _KC)

            nxt = k + 2

            @pl.when(nxt < nkc)
            def _():
                x_copy(nxt, b).start()
                w_copy(nxt, b).start()
        return 0

    lax.fori_loop(0, nkc // 2, outer, 0)

    xt_copy.wait()
    wt_copy.wait()
    accum_chunk(tb, wtb, _TAIL)

    b_copy.wait()
    lanes = lax.iota(jnp.int32, _LANES)

    def shuffle(v, idx):
        return lax.gather(
            v, idx[:, None],
            lax.GatherDimensionNumbers(
                offset_dims=(), collapsed_slice_dims=(0,),
                start_index_map=(0,)),
            slice_sizes=(1,),
            mode=lax.GatherScatterMode.PROMISE_IN_BOUNDS)

    def lane_sum(v):
        for s in (8, 4, 2, 1):
            v = v + shuffle(v, jnp.bitwise_xor(lanes, s))
        return v

    bias_full = shuffle(bv[...], jnp.bitwise_xor(lanes, lanes))
    for g in range(rows // _LANES):
        resv = jnp.zeros((_LANES,), jnp.float32)
        for j in range(_LANES):
            resv = jnp.where(lanes == j, lane_sum(acc[g * _LANES + j]), resv)
        res[pl.ds(g * _LANES, _LANES)] = resv + bias_full
    pltpu.sync_copy(res, o_hbm.at[pl.ds(base, rows)])


@jax.jit
def kernel(x, weight, bias):
    batch, vocab = x.shape
    info = plsc.get_sparse_core_info()
    nc, ns = info.num_cores, info.num_subcores
    nw = nc * ns
    rows = batch // nw
    nkc = vocab // _KC
    head = nkc * _KC
    tail = vocab - head

    wflat = weight.reshape(-1)
    xt = jnp.pad(x[:, head:], ((0, 0), (0, _TAIL - tail)))
    wt = jnp.pad(wflat[head:], (0, _TAIL - tail))
    bpad = jnp.pad(bias.reshape(-1), (0, _LANES - 1))

    mesh = plsc.VectorSubcoreMesh(core_axis_name="c", subcore_axis_name="s")
    body = functools.partial(_mv_body, rows=rows, nkc=nkc, nc=nc)
    fn = pl.kernel(
        body,
        mesh=mesh,
        out_type=jax.ShapeDtypeStruct((batch,), jnp.float32),
        scratch_types=[
            pltpu.VMEM((rows, _KC), jnp.float32),
            pltpu.VMEM((rows, _KC), jnp.float32),
            pltpu.VMEM((rows, _TAIL), jnp.float32),
            pltpu.VMEM((_KC,), jnp.float32),
            pltpu.VMEM((_KC,), jnp.float32),
            pltpu.VMEM((_TAIL,), jnp.float32),
            pltpu.VMEM((rows, _LANES), jnp.float32),
            pltpu.VMEM((rows,), jnp.float32),
            pltpu.VMEM((_LANES,), jnp.float32),
            pltpu.SemaphoreType.DMA,
            pltpu.SemaphoreType.DMA,
            pltpu.SemaphoreType.DMA,
            pltpu.SemaphoreType.DMA,
        ],
    )
    out = fn(x, xt, wflat[:head], wt, bpad)
    return out.reshape(batch, 1)
